# baseline (speedup 1.0000x reference)
import functools

import jax
import jax.numpy as jnp
from jax import lax
from jax.experimental import pallas as pl
from jax.experimental.pallas import tpu as pltpu
from jax.experimental.pallas import tpu_sc as plsc

BATCH = 1024
SEQ = 50
NROWS = BATCH * SEQ
HIDDEN = 64
NW = 32
ROWS_PER_W = NROWS // NW
RB = 3200


CHUNK = 800


def _gather_embeddings(len2, ipd2, pidx_len, pidx_ipd):
    mesh = plsc.VectorSubcoreMesh(core_axis_name="c", subcore_axis_name="s")

    @functools.partial(
        pl.kernel,
        mesh=mesh,
        out_type=(
            jax.ShapeDtypeStruct((NROWS, 2 * HIDDEN), jnp.float32),
            jax.ShapeDtypeStruct((NROWS, 2 * HIDDEN), jnp.float32),
        ),
        scratch_types=[
            pltpu.VMEM((CHUNK,), jnp.int32),
            pltpu.VMEM((CHUNK, 2 * HIDDEN), jnp.float32),
            pltpu.SemaphoreType.DMA,
        ],
    )
    def k(len_hbm, ipd_hbm, il_hbm, ii_hbm, out_l, out_i, idx_v, rows_v, sem):
        wid = lax.axis_index("s") * 2 + lax.axis_index("c")
        base0 = wid * ROWS_PER_W
        for t_hbm, i_hbm, o_hbm in ((len_hbm, il_hbm, out_l),
                                    (ipd_hbm, ii_hbm, out_i)):
            for ch in range(ROWS_PER_W // CHUNK):
                base = base0 + ch * CHUNK
                pltpu.sync_copy(i_hbm.at[pl.ds(base, CHUNK)], idx_v)
                pltpu.async_copy(t_hbm.at[idx_v], rows_v, sem).wait()
                pltpu.sync_copy(rows_v, o_hbm.at[pl.ds(base, CHUNK)])

    return k(len2, ipd2, pidx_len, pidx_ipd)


def _pq_half(v, S, H, T, LUT, acc0=None):
    ohs = []
    for c in range(8):
        p = jnp.dot(v[:, c * 8:(c + 1) * 8], S[c]) - T[c].reshape(1, 15)
        s = jnp.where(p > 0, 1.0, -1.0)
        l = jnp.dot(s, H)
        iot = lax.broadcasted_iota(jnp.int32, l.shape, 1)
        m = jnp.max(l, axis=-1, keepdims=True)
        cand = jnp.where(l == m, iot, l.shape[-1])
        idx = jnp.min(cand, axis=-1)
        ohs.append((idx[:, None] == iot).astype(jnp.float32))
    oh_all = jnp.concatenate(ohs, axis=1)
    r = jnp.dot(oh_all, LUT.reshape(8 * 16, -1))
    return r if acc0 is None else acc0 + r


def _stageA_body(lg_ref, ig_ref, parl_ref, pari_ref, S_ref, H_ref, T_ref,
                 LUT_ref, A_ref):
    lg = lg_ref[...]
    ig = ig_ref[...]
    len_sel = jnp.where(parl_ref[...] == 1, lg[:, HIDDEN:], lg[:, :HIDDEN])
    ipd_sel = jnp.where(pari_ref[...] == 1, ig[:, HIDDEN:], ig[:, :HIDDEN])
    xe = ipd_sel + len_sel
    A_ref[...] = _pq_half(xe, S_ref[...], H_ref[...], T_ref[...], LUT_ref[...])


def _stageA(lg, ig, parl, pari, S1x, H1, T1x, LUT1x):
    grid = NROWS // RB
    return pl.pallas_call(
        _stageA_body,
        grid=(grid,),
        in_specs=[
            pl.BlockSpec((RB, 2 * HIDDEN), lambda i: (i, 0)),
            pl.BlockSpec((RB, 2 * HIDDEN), lambda i: (i, 0)),
            pl.BlockSpec((RB, 1), lambda i: (i, 0)),
            pl.BlockSpec((RB, 1), lambda i: (i, 0)),
            pl.BlockSpec((8, 8, 15), lambda i: (0, 0, 0)),
            pl.BlockSpec((15, 16), lambda i: (0, 0)),
            pl.BlockSpec((8, 15), lambda i: (0, 0)),
            pl.BlockSpec((8, 16, HIDDEN), lambda i: (0, 0, 0)),
        ],
        out_specs=pl.BlockSpec((RB, HIDDEN), lambda i: (i, 0)),
        out_shape=jax.ShapeDtypeStruct((NROWS, HIDDEN), jnp.float32),
    )(lg, ig, parl, pari, S1x, H1, T1x, LUT1x)


def _stageC_body(A_ref, S1h_ref, H1_ref, T1h_ref, LUT1h_ref,
                 S2_ref, H2_ref, T2_ref, LUT2_ref, out_ref, h_ref):
    S1h = S1h_ref[...]
    H1 = H1_ref[...]
    T1h = T1h_ref[...]
    L1h = LUT1h_ref[...]
    h_ref[...] = jnp.zeros((BATCH, HIDDEN), jnp.float32)

    def step(t, carry):
        a = A_ref[t]
        h_ref[...] = _pq_half(h_ref[...], S1h, H1, T1h, L1h, acc0=a)
        return carry

    lax.fori_loop(0, SEQ, step, 0)

    o = _pq_half(h_ref[...], S2_ref[...], H2_ref[...], T2_ref[...],
                 LUT2_ref[...])
    m = jnp.max(o, axis=-1, keepdims=True)
    sh = o - m
    out_ref[...] = sh - jnp.log(jnp.sum(jnp.exp(sh), axis=-1, keepdims=True))


def _stageC(A, S1h, H1, T1h, LUT1h, S2, H2, T2, LUT2):
    return pl.pallas_call(
        _stageC_body,
        in_specs=[
            pl.BlockSpec((SEQ, BATCH, HIDDEN), lambda: (0, 0, 0)),
            pl.BlockSpec((8, 8, 15), lambda: (0, 0, 0)),
            pl.BlockSpec((15, 16), lambda: (0, 0)),
            pl.BlockSpec((8, 15), lambda: (0, 0)),
            pl.BlockSpec((8, 16, HIDDEN), lambda: (0, 0, 0)),
            pl.BlockSpec((8, 8, 15), lambda: (0, 0, 0)),
            pl.BlockSpec((15, 16), lambda: (0, 0)),
            pl.BlockSpec((8, 15), lambda: (0, 0)),
            pl.BlockSpec((8, 16, 100), lambda: (0, 0, 0)),
        ],
        out_specs=pl.BlockSpec((BATCH, 100), lambda: (0, 0)),
        out_shape=jax.ShapeDtypeStruct((BATCH, 100), jnp.float32),
        scratch_shapes=[pltpu.VMEM((BATCH, HIDDEN), jnp.float32)],
    )(A, S1h, H1, T1h, LUT1h, S2, H2, T2, LUT2)


def kernel(x, S1, H1, T1, LUT1, S2, H2, T2, LUT2, lenLUT, ipdLUT):
    xt = x.astype(jnp.int32).transpose(1, 0, 2)
    idx_len = xt[:, :, 0].reshape(NROWS)
    idx_ipd = xt[:, :, 1].reshape(NROWS)

    len2 = lenLUT.reshape(-1, 2 * HIDDEN)
    ipd2 = ipdLUT.reshape(-1, 2 * HIDDEN)
    lg, ig = _gather_embeddings(len2, ipd2, idx_len >> 1, idx_ipd >> 1)
    parl = (idx_len & 1).reshape(NROWS, 1)
    pari = (idx_ipd & 1).reshape(NROWS, 1)

    A = _stageA(lg, ig, parl, pari, S1[:8], H1, T1[:8], LUT1[:8])
    A = A.reshape(SEQ, BATCH, HIDDEN)

    return _stageC(A, S1[8:], H1, T1[8:], LUT1[8:], S2, H2, T2, LUT2)


# speedup vs baseline: 2.9228x; 2.9228x over previous
import functools

import jax
import jax.numpy as jnp
from jax import lax
from jax.experimental import pallas as pl
from jax.experimental.pallas import tpu as pltpu
from jax.experimental.pallas import tpu_sc as plsc

BATCH = 1024
SEQ = 50
NROWS = BATCH * SEQ
HIDDEN = 64
NW = 32
ROWS_PER_W = NROWS // NW
CHUNK = 800
RB = 3200


def _gather_embeddings(len2, ipd2, pidx_len, pidx_ipd):
    mesh = plsc.VectorSubcoreMesh(core_axis_name="c", subcore_axis_name="s")

    @functools.partial(
        pl.kernel,
        mesh=mesh,
        out_type=(
            jax.ShapeDtypeStruct((NROWS, 2 * HIDDEN), jnp.float32),
            jax.ShapeDtypeStruct((NROWS, 2 * HIDDEN), jnp.float32),
        ),
        scratch_types=[
            pltpu.VMEM((CHUNK,), jnp.int32),
            pltpu.VMEM((CHUNK, 2 * HIDDEN), jnp.float32),
            pltpu.SemaphoreType.DMA,
        ],
    )
    def k(len_hbm, ipd_hbm, il_hbm, ii_hbm, out_l, out_i, idx_v, rows_v, sem):
        wid = lax.axis_index("s") * 2 + lax.axis_index("c")
        base0 = wid * ROWS_PER_W
        for t_hbm, i_hbm, o_hbm in ((len_hbm, il_hbm, out_l),
                                    (ipd_hbm, ii_hbm, out_i)):
            for ch in range(ROWS_PER_W // CHUNK):
                base = base0 + ch * CHUNK
                pltpu.sync_copy(i_hbm.at[pl.ds(base, CHUNK)], idx_v)
                pltpu.async_copy(t_hbm.at[idx_v], rows_v, sem).wait()
                pltpu.sync_copy(rows_v, o_hbm.at[pl.ds(base, CHUNK)])

    return k(len2, ipd2, pidx_len, pidx_ipd)


def _build_mats(S, H, T, LUT):
    eye = jnp.eye(8, dtype=jnp.float32)
    St = jnp.pad(jnp.transpose(S, (0, 2, 1)), ((0, 0), (0, 1), (0, 0)))
    M1 = (eye[:, None, :, None] * St[:, :, None, :]).reshape(128, 64)
    Tc = jnp.pad(T, ((0, 0), (0, 1))).reshape(128, 1)
    Hp = jnp.pad(H.T, ((0, 0), (0, 1)))
    M2 = (eye[:, None, :, None] * Hp[None, :, None, :]).reshape(128, 128)
    M3 = jnp.transpose(LUT, (2, 0, 1)).reshape(LUT.shape[2], 128)
    return M1, Tc, M2, M3


def _pq_stage(vT_or_v, M1, Tc, M2, n, transposed_in=True):
    if transposed_in:
        pT = jnp.dot(M1, vT_or_v)
    else:
        pT = lax.dot_general(M1, vT_or_v, (((1,), (1,)), ((), ())))
    sT = jnp.where(pT - Tc > 0, 1.0, -1.0)
    lT = jnp.dot(M2, sT)
    v = lT.reshape(8, 16, n)
    m = v
    for hw in (8, 4, 2, 1):
        m = jnp.maximum(m[:, :hw, :], m[:, hw:2 * hw, :])
    iot = lax.broadcasted_iota(jnp.int32, (8, 16, n), 1)
    cand = jnp.where(v == m, iot, 16)
    mi = cand
    for hw in (8, 4, 2, 1):
        mi = jnp.minimum(mi[:, :hw, :], mi[:, hw:2 * hw, :])
    return (iot == mi).astype(jnp.float32).reshape(128, n)


def _stageA_body(lg_ref, ig_ref, parl_ref, pari_ref, M1_ref, Tc_ref, M2_ref,
                 M3_ref, AT_ref):
    lgv = lg_ref[...]
    igv = ig_ref[...]
    len_sel = jnp.where(parl_ref[...] == 1, lgv[:, HIDDEN:], lgv[:, :HIDDEN])
    ipd_sel = jnp.where(pari_ref[...] == 1, igv[:, HIDDEN:], igv[:, :HIDDEN])
    xe = ipd_sel + len_sel
    oh = _pq_stage(xe, M1_ref[...], Tc_ref[...], M2_ref[...], RB,
                   transposed_in=False)
    AT_ref[...] = jnp.dot(M3_ref[...], oh)


def _stageA(lg, ig, parl, pari, M1, Tc, M2, M3):
    grid = NROWS // RB
    return pl.pallas_call(
        _stageA_body,
        grid=(grid,),
        in_specs=[
            pl.BlockSpec((RB, 2 * HIDDEN), lambda i: (i, 0)),
            pl.BlockSpec((RB, 2 * HIDDEN), lambda i: (i, 0)),
            pl.BlockSpec((RB, 1), lambda i: (i, 0)),
            pl.BlockSpec((RB, 1), lambda i: (i, 0)),
            pl.BlockSpec((128, 64), lambda i: (0, 0)),
            pl.BlockSpec((128, 1), lambda i: (0, 0)),
            pl.BlockSpec((128, 128), lambda i: (0, 0)),
            pl.BlockSpec((64, 128), lambda i: (0, 0)),
        ],
        out_specs=pl.BlockSpec((64, RB), lambda i: (0, i)),
        out_shape=jax.ShapeDtypeStruct((64, NROWS), jnp.float32),
    )(lg, ig, parl, pari, M1, Tc, M2, M3)


def _stageC_body(AT_ref, M1h_ref, Tch_ref, M2h_ref, M3h_ref,
                 M1f_ref, Tcf_ref, M2f_ref, M3f_ref, out_ref, hT_ref):
    M1h = M1h_ref[...]
    Tch = Tch_ref[...]
    M2h = M2h_ref[...]
    M3h = M3h_ref[...]
    hT_ref[...] = jnp.zeros((HIDDEN, BATCH), jnp.float32)

    def step(t, carry):
        at = AT_ref[:, pl.ds(t * BATCH, BATCH)]
        oh = _pq_stage(hT_ref[...], M1h, Tch, M2h, BATCH)
        hT_ref[...] = at + jnp.dot(M3h, oh)
        return carry

    lax.fori_loop(0, SEQ, step, 0)

    ohf = _pq_stage(hT_ref[...], M1f_ref[...], Tcf_ref[...], M2f_ref[...],
                    BATCH)
    o = lax.dot_general(ohf, M3f_ref[...], (((0,), (1,)), ((), ())))
    m = jnp.max(o, axis=-1, keepdims=True)
    sh = o - m
    out_ref[...] = sh - jnp.log(jnp.sum(jnp.exp(sh), axis=-1, keepdims=True))


def _stageC(AT, mats_h, mats_f):
    return pl.pallas_call(
        _stageC_body,
        in_specs=[
            pl.BlockSpec((HIDDEN, NROWS), lambda: (0, 0)),
            pl.BlockSpec((128, 64), lambda: (0, 0)),
            pl.BlockSpec((128, 1), lambda: (0, 0)),
            pl.BlockSpec((128, 128), lambda: (0, 0)),
            pl.BlockSpec((64, 128), lambda: (0, 0)),
            pl.BlockSpec((128, 64), lambda: (0, 0)),
            pl.BlockSpec((128, 1), lambda: (0, 0)),
            pl.BlockSpec((128, 128), lambda: (0, 0)),
            pl.BlockSpec((100, 128), lambda: (0, 0)),
        ],
        out_specs=pl.BlockSpec((BATCH, 100), lambda: (0, 0)),
        out_shape=jax.ShapeDtypeStruct((BATCH, 100), jnp.float32),
        scratch_shapes=[pltpu.VMEM((HIDDEN, BATCH), jnp.float32)],
    )(AT, *mats_h, *mats_f)


def kernel(x, S1, H1, T1, LUT1, S2, H2, T2, LUT2, lenLUT, ipdLUT):
    xt = x.astype(jnp.int32).transpose(1, 0, 2)
    idx_len = xt[:, :, 0].reshape(NROWS)
    idx_ipd = xt[:, :, 1].reshape(NROWS)

    len2 = lenLUT.reshape(-1, 2 * HIDDEN)
    ipd2 = ipdLUT.reshape(-1, 2 * HIDDEN)
    lg, ig = _gather_embeddings(len2, ipd2, idx_len >> 1, idx_ipd >> 1)
    parl = (idx_len & 1).reshape(NROWS, 1)
    pari = (idx_ipd & 1).reshape(NROWS, 1)

    mats_x = _build_mats(S1[:8], H1, T1[:8], LUT1[:8])
    mats_h = _build_mats(S1[8:], H1, T1[8:], LUT1[8:])
    mats_f = _build_mats(S2, H2, T2, LUT2)

    AT = _stageA(lg, ig, parl, pari, *mats_x)
    return _stageC(AT, mats_h, mats_f)


# speedup vs baseline: 3.0058x; 1.0284x over previous
import functools

import jax
import jax.numpy as jnp
from jax import lax
from jax.experimental import pallas as pl
from jax.experimental.pallas import tpu as pltpu
from jax.experimental.pallas import tpu_sc as plsc

BATCH = 1024
SEQ = 50
NROWS = BATCH * SEQ
HIDDEN = 64
NW = 32
ROWS_PER_W = NROWS // NW
CHUNK = 800
RB = 3200


def _gather_embeddings(len2, ipd2, pidx_len, pidx_ipd):
    mesh = plsc.VectorSubcoreMesh(core_axis_name="c", subcore_axis_name="s")

    @functools.partial(
        pl.kernel,
        mesh=mesh,
        out_type=(
            jax.ShapeDtypeStruct((NROWS, 2 * HIDDEN), jnp.float32),
            jax.ShapeDtypeStruct((NROWS, 2 * HIDDEN), jnp.float32),
        ),
        scratch_types=[
            pltpu.VMEM((CHUNK,), jnp.int32),
            pltpu.VMEM((CHUNK, 2 * HIDDEN), jnp.float32),
            pltpu.SemaphoreType.DMA,
        ],
    )
    def k(len_hbm, ipd_hbm, il_hbm, ii_hbm, out_l, out_i, idx_v, rows_v, sem):
        wid = lax.axis_index("s") * 2 + lax.axis_index("c")
        base0 = wid * ROWS_PER_W
        for t_hbm, i_hbm, o_hbm in ((len_hbm, il_hbm, out_l),
                                    (ipd_hbm, ii_hbm, out_i)):
            for ch in range(ROWS_PER_W // CHUNK):
                base = base0 + ch * CHUNK
                pltpu.sync_copy(i_hbm.at[pl.ds(base, CHUNK)], idx_v)
                pltpu.async_copy(t_hbm.at[idx_v], rows_v, sem).wait()
                pltpu.sync_copy(rows_v, o_hbm.at[pl.ds(base, CHUNK)])

    return k(len2, ipd2, pidx_len, pidx_ipd)


def _build_mats(S, H, T, LUT):
    eye = jnp.eye(8, dtype=jnp.float32)
    St = jnp.pad(jnp.transpose(S, (0, 2, 1)), ((0, 0), (0, 1), (0, 0)))
    M1 = (eye[:, None, :, None] * St[:, :, None, :]).reshape(128, 64)
    Tc = jnp.pad(T, ((0, 0), (0, 1))).reshape(128, 1)
    Hp = jnp.pad(H.T, ((0, 0), (0, 1)))
    M2 = (eye[:, None, :, None] * Hp[None, :, None, :]).reshape(128, 128)
    M3 = jnp.transpose(LUT, (2, 0, 1)).reshape(LUT.shape[2], 128)
    return M1, Tc, M2, M3


def _pq_stage(vT_or_v, M1, Tc, M2, n, transposed_in=True):
    if transposed_in:
        pT = jnp.dot(M1, vT_or_v)
    else:
        pT = lax.dot_general(M1, vT_or_v, (((1,), (1,)), ((), ())))
    sT = jnp.where(pT - Tc > 0, 1.0, -1.0)
    lT = jnp.dot(M2, sT)
    v = lT.reshape(8, 16, n)
    m = v
    for hw in (8, 4, 2, 1):
        m = jnp.maximum(m[:, :hw, :], m[:, hw:2 * hw, :])
    iot = lax.broadcasted_iota(jnp.int32, (8, 16, n), 1)
    cand = jnp.where(v == m, iot, 16)
    mi = cand
    for hw in (8, 4, 2, 1):
        mi = jnp.minimum(mi[:, :hw, :], mi[:, hw:2 * hw, :])
    return (iot == mi).astype(jnp.float32).reshape(128, n)


RBF = 5 * BATCH
GRID = NROWS // RBF


def _fused_body(lg_ref, ig_ref, parl_ref, pari_ref,
                Mx1_ref, Txc_ref, Mx2_ref, Mx3_ref,
                Mh1_ref, Thc_ref, Mh2_ref, Mh3_ref,
                Mf1_ref, Tfc_ref, Mf2_ref, Mf3_ref,
                out_ref, hT_ref, AT_ref):
    i = pl.program_id(0)

    @pl.when(i == 0)
    def _():
        hT_ref[...] = jnp.zeros((HIDDEN, BATCH), jnp.float32)

    lgv = lg_ref[...]
    igv = ig_ref[...]
    len_sel = jnp.where(parl_ref[...] == 1, lgv[:, HIDDEN:], lgv[:, :HIDDEN])
    ipd_sel = jnp.where(pari_ref[...] == 1, igv[:, HIDDEN:], igv[:, :HIDDEN])
    xe = ipd_sel + len_sel
    oh = _pq_stage(xe, Mx1_ref[...], Txc_ref[...], Mx2_ref[...], RBF,
                   transposed_in=False)
    AT_ref[...] = jnp.dot(Mx3_ref[...], oh)

    Mh1 = Mh1_ref[...]
    Thc = Thc_ref[...]
    Mh2 = Mh2_ref[...]
    Mh3 = Mh3_ref[...]
    for k in range(RBF // BATCH):
        at = AT_ref[:, k * BATCH:(k + 1) * BATCH]
        ohk = _pq_stage(hT_ref[...], Mh1, Thc, Mh2, BATCH)
        hT_ref[...] = at + jnp.dot(Mh3, ohk)

    @pl.when(i == GRID - 1)
    def _():
        ohf = _pq_stage(hT_ref[...], Mf1_ref[...], Tfc_ref[...], Mf2_ref[...],
                        BATCH)
        o = lax.dot_general(ohf, Mf3_ref[...], (((0,), (1,)), ((), ())))
        m = jnp.max(o, axis=-1, keepdims=True)
        sh = o - m
        out_ref[...] = sh - jnp.log(jnp.sum(jnp.exp(sh), axis=-1,
                                            keepdims=True))


def _fused(lg, ig, parl, pari, mats_x, mats_h, mats_f):
    wspec = [
        pl.BlockSpec((128, 64), lambda i: (0, 0)),
        pl.BlockSpec((128, 1), lambda i: (0, 0)),
        pl.BlockSpec((128, 128), lambda i: (0, 0)),
    ]
    return pl.pallas_call(
        _fused_body,
        grid=(GRID,),
        in_specs=[
            pl.BlockSpec((RBF, 2 * HIDDEN), lambda i: (i, 0)),
            pl.BlockSpec((RBF, 2 * HIDDEN), lambda i: (i, 0)),
            pl.BlockSpec((RBF, 1), lambda i: (i, 0)),
            pl.BlockSpec((RBF, 1), lambda i: (i, 0)),
        ] + wspec + [pl.BlockSpec((64, 128), lambda i: (0, 0))]
          + wspec + [pl.BlockSpec((64, 128), lambda i: (0, 0))]
          + wspec + [pl.BlockSpec((100, 128), lambda i: (0, 0))],
        out_specs=pl.BlockSpec((BATCH, 100), lambda i: (0, 0)),
        out_shape=jax.ShapeDtypeStruct((BATCH, 100), jnp.float32),
        scratch_shapes=[pltpu.VMEM((HIDDEN, BATCH), jnp.float32),
                        pltpu.VMEM((HIDDEN, RBF), jnp.float32)],
    )(lg, ig, parl, pari, *mats_x, *mats_h, *mats_f)


def kernel(x, S1, H1, T1, LUT1, S2, H2, T2, LUT2, lenLUT, ipdLUT):
    xt = x.astype(jnp.int32).transpose(1, 0, 2)
    idx_len = xt[:, :, 0].reshape(NROWS)
    idx_ipd = xt[:, :, 1].reshape(NROWS)

    len2 = lenLUT.reshape(-1, 2 * HIDDEN)
    ipd2 = ipdLUT.reshape(-1, 2 * HIDDEN)
    lg, ig = _gather_embeddings(len2, ipd2, idx_len >> 1, idx_ipd >> 1)
    parl = (idx_len & 1).reshape(NROWS, 1)
    pari = (idx_ipd & 1).reshape(NROWS, 1)

    mats_x = _build_mats(S1[:8], H1, T1[:8], LUT1[:8])
    mats_h = _build_mats(S1[8:], H1, T1[8:], LUT1[8:])
    mats_f = _build_mats(S2, H2, T2, LUT2)

    return _fused(lg, ig, parl, pari, mats_x, mats_h, mats_f)
